# baseline (speedup 1.0000x reference)
import functools

import jax
import jax.numpy as jnp
from jax import lax
from jax.experimental import pallas as pl
from jax.experimental.pallas import tpu as pltpu
from jax.experimental.pallas import tpu_sc as plsc

_NPOINT = 512
_RADII = (0.1, 0.2, 0.4)
_NSAMPLES = (16, 32, 64)
_TABLE_D = 128


def _sc_gather(table, gidx):
    R = gidx.shape[0]
    D = table.shape[1]
    NC, NS = 2, 16
    NW = NC * NS
    CH = 128
    per_w = R // NW
    n_ch = per_w // CH
    mesh = plsc.VectorSubcoreMesh(core_axis_name="c", subcore_axis_name="s")

    @functools.partial(
        pl.kernel, mesh=mesh,
        out_type=jax.ShapeDtypeStruct((R, D), jnp.float32),
        scratch_types=[
            pltpu.VMEM((CH,), jnp.int32),
            pltpu.VMEM((CH, D), jnp.float32),
            pltpu.SemaphoreType.DMA,
        ],
    )
    def k(table_hbm, gidx_hbm, out_hbm, idx_v, rows_v, sem):
        wid = lax.axis_index("s") * NC + lax.axis_index("c")
        base = wid * per_w

        def body(i, _):
            off = base + i * CH
            pltpu.sync_copy(gidx_hbm.at[pl.ds(off, CH)], idx_v)
            pltpu.async_copy(table_hbm.at[idx_v], rows_v, sem).wait()
            pltpu.sync_copy(rows_v, out_hbm.at[pl.ds(off, CH)])
            return 0

        lax.fori_loop(0, n_ch, body, 0)

    return k(table, gidx)


def _fps_body(xyz_ref, out_ref):
    B, _, N = xyz_ref.shape
    x = xyz_ref[:, 0, :]
    y = xyz_ref[:, 1, :]
    z = xyz_ref[:, 2, :]
    iota = jax.lax.broadcasted_iota(jnp.int32, (B, N), 1)

    def step(i, carry):
        dist, far = carry
        out_ref[pl.ds(i, 1), :] = far.T
        sel = iota == far
        cx = jnp.sum(jnp.where(sel, x, 0.0), axis=1, keepdims=True)
        cy = jnp.sum(jnp.where(sel, y, 0.0), axis=1, keepdims=True)
        cz = jnp.sum(jnp.where(sel, z, 0.0), axis=1, keepdims=True)
        dx = x - cx
        dy = y - cy
        dz = z - cz
        d = dx * dx + dy * dy + dz * dz
        dist = jnp.minimum(dist, d)
        m = jnp.max(dist, axis=1, keepdims=True)
        far_new = jnp.min(jnp.where(dist == m, iota, N), axis=1, keepdims=True)
        return dist, far_new.astype(jnp.int32)

    dist0 = jnp.full((B, N), 1e10, dtype=jnp.float32)
    far0 = jnp.zeros((B, 1), dtype=jnp.int32)
    jax.lax.fori_loop(0, out_ref.shape[0], step, (dist0, far0))


def _fps(xyz):
    B, _, N = xyz.shape
    out = pl.pallas_call(
        _fps_body,
        out_shape=jax.ShapeDtypeStruct((_NPOINT, B), jnp.int32),
        in_specs=[pl.BlockSpec(memory_space=pltpu.MemorySpace.VMEM)],
        out_specs=pl.BlockSpec(memory_space=pltpu.MemorySpace.VMEM),
    )(xyz)
    return out.T


def _index_points(points, idx):
    return jax.vmap(lambda p, i: p[i])(points, idx)


def kernel(xyz, points, params):
    B, _, N = xyz.shape
    S = _NPOINT
    xyz_t = jnp.transpose(xyz, (0, 2, 1))
    pts_t = jnp.transpose(points, (0, 2, 1))

    fps_idx = _fps(xyz)
    new_xyz = _index_points(xyz_t, fps_idx)

    table = jnp.concatenate(
        [pts_t, xyz_t, jnp.zeros((B, N, _TABLE_D - 67), jnp.float32)],
        axis=-1).reshape(B * N, _TABLE_D)

    d = -2.0 * jnp.einsum('bsc,bnc->bsn', new_xyz, xyz_t)
    d = d + jnp.sum(new_xyz ** 2, axis=-1)[:, :, None]
    d = d + jnp.sum(xyz_t ** 2, axis=-1)[:, None, :]

    idx_flat = []
    for r, K in zip(_RADII, _NSAMPLES):
        mask = d <= r * r
        C = jnp.cumsum(mask.astype(jnp.int32), axis=-1)
        ks = jnp.arange(K, dtype=jnp.int32)
        idx = jax.vmap(jax.vmap(
            lambda row: jnp.searchsorted(row, ks, side='right')))(C)
        idx = jnp.where(idx == N, idx[..., :1], idx).astype(jnp.int32)
        gidx = idx + (jnp.arange(B, dtype=jnp.int32) * N)[:, None, None]
        idx_flat.append(gidx.reshape(-1))
    gathered = _sc_gather(table, jnp.concatenate(idx_flat))

    outs = []
    split = 0
    for r, K in zip(_RADII, _NSAMPLES):
        g_rows = gathered[split:split + B * S * K]
        split += B * S * K
        g_rows = g_rows.reshape(B, S, K, _TABLE_D)
        g_pts = g_rows[..., :64]
        g_xyz = g_rows[..., 64:67] - new_xyz[:, :, None, :]
        g = jnp.concatenate([g_pts, g_xyz], axis=-1)
        g = jnp.transpose(g, (0, 3, 2, 1))
        for layer in params[len(outs)]:
            g = jnp.einsum('oc,bcks->boks', layer["W"], g) + layer["b"][None, :, None, None]
            mean = jnp.mean(g, axis=(0, 2, 3), keepdims=True)
            var = jnp.var(g, axis=(0, 2, 3), keepdims=True)
            g = (g - mean) / jnp.sqrt(var + 1e-5)
            g = g * layer["gamma"][None, :, None, None] + layer["beta"][None, :, None, None]
            g = jax.nn.relu(g)
        outs.append(jnp.max(g, axis=2))

    return (jnp.transpose(new_xyz, (0, 2, 1)), jnp.concatenate(outs, axis=1))


# speedup vs baseline: 1.7384x; 1.7384x over previous
import functools

import jax
import jax.numpy as jnp
from jax import lax
from jax.experimental import pallas as pl
from jax.experimental.pallas import tpu as pltpu
from jax.experimental.pallas import tpu_sc as plsc

_NPOINT = 512
_RADII = (0.1, 0.2, 0.4)
_NSAMPLES = (16, 32, 64)
_TABLE_D = 128


_SBLK = 128


def _select_body(q_ref, p_ref, e1_ref, e2_ref, e3_ref):
    N = p_ref.shape[2]
    q = q_ref[0]
    p = p_ref[0]
    qp = jax.lax.dot_general(q, p, (((1,), (0,)), ((), ())),
                             preferred_element_type=jnp.float32)
    q2 = jnp.sum(q * q, axis=1, keepdims=True)
    p2 = jnp.sum(p * p, axis=0, keepdims=True)
    d = -2.0 * qp + q2 + p2

    masks = [(d <= r * r).astype(jnp.float32) for r in _RADII]
    m_all = jnp.concatenate(masks, axis=0)

    it = jax.lax.broadcasted_iota(jnp.int32, (128, 128), 0)
    jt = jax.lax.broadcasted_iota(jnp.int32, (128, 128), 1)
    T = (it <= jt).astype(jnp.float32)

    carry = jnp.zeros((3 * _SBLK, 1), jnp.float32)
    NW = N // 4
    for g in range(N // 128):
        seg = m_all[:, g * 128:(g + 1) * 128]
        loc = jax.lax.dot_general(seg, T, (((1,), (0,)), ((), ())),
                                  preferred_element_type=jnp.float32)
        C = loc + carry
        carry = carry + loc[:, 127:128]
        byte = g // (N // 512)
        woff = (g % (N // 512)) * 128
        for ri, (e_ref, K) in enumerate(
                zip((e1_ref, e2_ref, e3_ref), _NSAMPLES)):
            Cg = C[ri * _SBLK:(ri + 1) * _SBLK]
            mg = m_all[ri * _SBLK:(ri + 1) * _SBLK, g * 128:(g + 1) * 128]
            e = jnp.where(mg > 0.0,
                          jnp.minimum(Cg, float(K + 1)), 0.0).astype(jnp.int32)
            word = e << (8 * byte)
            if byte == 0:
                e_ref[0, :, woff:woff + 128] = word
            else:
                e_ref[0, :, woff:woff + 128] = e_ref[0, :, woff:woff + 128] | word


def _select(new_xyz, xyz):
    B, _, N = xyz.shape
    S = _NPOINT
    grid = (B, S // _SBLK)
    outs = pl.pallas_call(
        _select_body,
        grid=grid,
        in_specs=[
            pl.BlockSpec((1, _SBLK, 3), lambda b, s: (b, s, 0)),
            pl.BlockSpec((1, 3, N), lambda b, s: (b, 0, 0)),
        ],
        out_specs=[pl.BlockSpec((1, _SBLK, N // 4), lambda b, s: (b, s, 0))
                   for _ in range(3)],
        out_shape=[jax.ShapeDtypeStruct((B, S, N // 4), jnp.int32)
                   for _ in range(3)],
    )(new_xyz, xyz)
    return [o.reshape(B * S, N // 4) for o in outs]


def _sc_extract_gather(table, e_packed, K, N):
    RQ, NWRD = e_packed.shape
    NC, NS = 2, 16
    rows_per_w = RQ // (NC * NS)
    mesh = plsc.VectorSubcoreMesh(core_axis_name="c", subcore_axis_name="s")

    @functools.partial(
        pl.kernel, mesh=mesh,
        compiler_params=pltpu.CompilerParams(needs_layout_passes=False),
        out_type=jax.ShapeDtypeStruct((RQ * K, 128), jnp.float32),
        scratch_types=[
            pltpu.VMEM((NWRD,), jnp.int32),
            pltpu.VMEM((8, 128), jnp.int32),
            pltpu.VMEM((K,), jnp.int32),
            pltpu.VMEM((K, 128), jnp.float32),
            pltpu.SemaphoreType.DMA,
        ],
    )
    def k(table_hbm, e_hbm, out_hbm, e_v, idx_v, gidx_v, rows_v, sem):
        wid = lax.axis_index("s") * NC + lax.axis_index("c")
        base = wid * rows_per_w
        lane = jax.lax.iota(jnp.int32, 16)
        zero16 = jnp.zeros((16,), jnp.int32)
        sentinel = jnp.full((16,), N, jnp.int32)

        def row_body(i, _):
            row = base + i
            pltpu.sync_copy(e_hbm.at[row], e_v)
            for g in range(K // 16):
                idx_v[0, pl.ds(g * 16, 16)] = sentinel

            def wchunk(c, _c):
                wv = e_v[pl.ds(c * 16, 16)]
                wbase = lane + c * 16
                for b in range(4):
                    eb = (wv >> (8 * b)) & 0xFF
                    valid = (eb > 0) & (eb <= K)
                    plsc.store_scatter(idx_v, [zero16, eb - 1],
                                       wbase + b * NWRD, mask=valid)
                return _c

            lax.fori_loop(0, NWRD // 16, wchunk, 0)

            first = jnp.broadcast_to(
                jnp.min(idx_v[0, pl.ds(0, 16)], axis=0), (16,))
            boff = jnp.full((16,), (row // _NPOINT) * N, jnp.int32)
            for g in range(K // 16):
                v = idx_v[0, pl.ds(g * 16, 16)]
                gidx_v[pl.ds(g * 16, 16)] = (
                    jnp.where(v == N, first, v) + boff)
            pltpu.async_copy(table_hbm.at[gidx_v], rows_v, sem).wait()
            pltpu.sync_copy(rows_v, out_hbm.at[pl.ds(row * K, K)])
            return _

        lax.fori_loop(0, rows_per_w, row_body, 0)

    return k(table, e_packed)


def _fps_body(xyz_ref, out_ref, nxyz_ref):
    B, _, N = xyz_ref.shape
    x = xyz_ref[:, 0, :]
    y = xyz_ref[:, 1, :]
    z = xyz_ref[:, 2, :]
    iota = jax.lax.broadcasted_iota(jnp.int32, (B, N), 1)

    def step(i, carry):
        dist, far = carry
        out_ref[pl.ds(i, 1), :] = far.T
        sel = iota == far
        cx = jnp.sum(jnp.where(sel, x, 0.0), axis=1, keepdims=True)
        cy = jnp.sum(jnp.where(sel, y, 0.0), axis=1, keepdims=True)
        cz = jnp.sum(jnp.where(sel, z, 0.0), axis=1, keepdims=True)
        nxyz_ref[pl.ds(i, 1), :, :] = jnp.concatenate(
            [cx, cy, cz], axis=1)[None, :, :]
        dx = x - cx
        dy = y - cy
        dz = z - cz
        d = dx * dx + dy * dy + dz * dz
        dist = jnp.minimum(dist, d)
        m = jnp.max(dist, axis=1, keepdims=True)
        far_new = jnp.min(jnp.where(dist == m, iota, N), axis=1, keepdims=True)
        return dist, far_new.astype(jnp.int32)

    dist0 = jnp.full((B, N), 1e10, dtype=jnp.float32)
    far0 = jnp.zeros((B, 1), dtype=jnp.int32)
    jax.lax.fori_loop(0, out_ref.shape[0], step, (dist0, far0))


def _fps(xyz):
    B, _, N = xyz.shape
    out, nxyz = pl.pallas_call(
        _fps_body,
        out_shape=[jax.ShapeDtypeStruct((_NPOINT, B), jnp.int32),
                   jax.ShapeDtypeStruct((_NPOINT, B, 3), jnp.float32)],
        in_specs=[pl.BlockSpec(memory_space=pltpu.MemorySpace.VMEM)],
        out_specs=[pl.BlockSpec(memory_space=pltpu.MemorySpace.VMEM),
                   pl.BlockSpec(memory_space=pltpu.MemorySpace.VMEM)],
    )(xyz)
    return out.T, jnp.transpose(nxyz, (1, 0, 2))


def _index_points(points, idx):
    return jax.vmap(lambda p, i: p[i])(points, idx)


def kernel(xyz, points, params):
    B, _, N = xyz.shape
    S = _NPOINT
    xyz_t = jnp.transpose(xyz, (0, 2, 1))
    pts_t = jnp.transpose(points, (0, 2, 1))

    _, new_xyz = _fps(xyz)

    table = jnp.concatenate(
        [pts_t, xyz_t, jnp.zeros((B, N, _TABLE_D - 67), jnp.float32)],
        axis=-1).reshape(B * N, _TABLE_D)

    e_packed = _select(new_xyz, xyz)

    outs = []
    for ri, (r, K) in enumerate(zip(_RADII, _NSAMPLES)):
        g_rows = _sc_extract_gather(table, e_packed[ri], K, N)
        g_rows = g_rows.reshape(B, S, K, _TABLE_D)
        g_pts = g_rows[..., :64]
        g_xyz = g_rows[..., 64:67] - new_xyz[:, :, None, :]
        g = jnp.concatenate([g_pts, g_xyz], axis=-1)
        g = jnp.transpose(g, (0, 3, 2, 1))
        for layer in params[len(outs)]:
            g = jnp.einsum('oc,bcks->boks', layer["W"], g) + layer["b"][None, :, None, None]
            mean = jnp.mean(g, axis=(0, 2, 3), keepdims=True)
            var = jnp.var(g, axis=(0, 2, 3), keepdims=True)
            g = (g - mean) / jnp.sqrt(var + 1e-5)
            g = g * layer["gamma"][None, :, None, None] + layer["beta"][None, :, None, None]
            g = jax.nn.relu(g)
        outs.append(jnp.max(g, axis=2))

    return (jnp.transpose(new_xyz, (0, 2, 1)), jnp.concatenate(outs, axis=1))
